# baseline (speedup 1.0000x reference)
import functools

import jax
import jax.numpy as jnp
from jax import lax
from jax.experimental import pallas as pl
from jax.experimental.pallas import tpu as pltpu
from jax.experimental.pallas import tpu_sc as plsc

T, D, E, F, C = 2048, 768, 64, 2048, 128
RC = 32
NRC = C // RC
FC = 1024
NF = F // FC

_NC, _NS, _L = 2, 16, 16
_NW = 16
_TPW = T // _NW


def _sc_argmax_body(logits_t_hbm, ids_hbm, blk_v, ids_v, sem):
    wid = lax.axis_index("s") * _NC + lax.axis_index("c")

    @pl.when(wid < _NW)
    def _():
        base = wid * _TPW
        pltpu.sync_copy(logits_t_hbm.at[:, pl.ds(base, _TPW)], blk_v)
        for g in range(_TPW // _L):
            best = blk_v[0, pl.ds(g * _L, _L)]
            bid = jnp.zeros((_L,), jnp.int32)
            for e in range(1, E):
                v = blk_v[e, pl.ds(g * _L, _L)]
                upd = v > best
                best = jnp.where(upd, v, best)
                bid = jnp.where(upd, jnp.full((_L,), e, jnp.int32), bid)
            ids_v[pl.ds(g * _L, _L)] = bid
        pltpu.sync_copy(ids_v, ids_hbm.at[pl.ds(base, _TPW)])


_sc_argmax = functools.partial(
    pl.kernel,
    out_type=jax.ShapeDtypeStruct((T,), jnp.int32),
    mesh=plsc.VectorSubcoreMesh(core_axis_name="c", subcore_axis_name="s",
                                num_cores=_NC, num_subcores=_NS),
    scratch_types=[
        pltpu.VMEM((E, _TPW), jnp.float32),
        pltpu.VMEM((_TPW,), jnp.int32),
        pltpu.SemaphoreType.DMA,
    ],
)(_sc_argmax_body)


def _argmax_body(logits_ref, ids_ref):
    ids_ref[...] = jnp.argmax(logits_ref[...], axis=1, keepdims=True).astype(jnp.int32)


def _ffn_body(ids_ref, hid_ref, wg_ref, wu_ref, wd_ref, out_ref,
              xb_ref, acc_ref, tfs_ref, cnt_ref):
    e = pl.program_id(0)
    f = pl.program_id(1)

    @pl.when(jnp.logical_and(e == 0, f == 0))
    def _():
        out_ref[...] = jnp.zeros_like(out_ref)

        def init_cnt(i, _):
            cnt_ref[i] = 0
            return 0
        jax.lax.fori_loop(0, E, init_cnt, 0, unroll=True)

        def rank(t, _):
            ex = ids_ref[t]
            p = cnt_ref[ex]

            @pl.when(p < C)
            def _():
                tfs_ref[ex * C + p] = t

            cnt_ref[ex] = p + 1
            return 0
        jax.lax.fori_loop(0, T, rank, 0)

    n = jnp.minimum(cnt_ref[e], C)

    @pl.when(f == 0)
    def _():
        xb_ref[...] = jnp.zeros_like(xb_ref)

        def gather(c, _):
            t = tfs_ref[e * C + c]
            xb_ref[pl.ds(c, 1), :] = hid_ref[pl.ds(t, 1), :]
            return 0
        jax.lax.fori_loop(0, n, gather, 0)

    for k in range(NRC):
        @pl.when(n > k * RC)
        def _(k=k):
            xs = xb_ref[k * RC:(k + 1) * RC, :]
            g = jnp.dot(xs, wg_ref[0], preferred_element_type=jnp.float32)
            u = jnp.dot(xs, wu_ref[0], preferred_element_type=jnp.float32)
            h = g * jax.nn.sigmoid(g) * u
            part = jnp.dot(h, wd_ref[0], preferred_element_type=jnp.float32)

            @pl.when(f == 0)
            def _():
                acc_ref[k * RC:(k + 1) * RC, :] = part

            @pl.when(f > 0)
            def _():
                acc_ref[k * RC:(k + 1) * RC, :] += part

    @pl.when(f == NF - 1)
    def _():
        def scatter(c, _):
            t = tfs_ref[e * C + c]
            out_ref[pl.ds(t, 1), :] = acc_ref[pl.ds(c, 1), :]
            return 0
        jax.lax.fori_loop(0, n, scatter, 0)


@functools.partial(jax.jit, static_argnames=("interpret",))
def kernel(hidden_states, router_logits, w_gate, w_up, w_down, interpret=False):
    if interpret:
        ids = pl.pallas_call(
            _argmax_body,
            out_shape=jax.ShapeDtypeStruct((T, 1), jnp.int32),
            interpret=True,
        )(router_logits).reshape(T)
    else:
        ids = _sc_argmax(router_logits.T)

    out = pl.pallas_call(
        _ffn_body,
        grid_spec=pltpu.PrefetchScalarGridSpec(
            num_scalar_prefetch=1,
            grid=(E, NF),
            in_specs=[
                pl.BlockSpec((T, D), lambda e, f, *_: (0, 0)),
                pl.BlockSpec((1, D, FC), lambda e, f, *_: (e, 0, f)),
                pl.BlockSpec((1, D, FC), lambda e, f, *_: (e, 0, f)),
                pl.BlockSpec((1, FC, D), lambda e, f, *_: (e, f, 0)),
            ],
            out_specs=pl.BlockSpec((T, D), lambda e, f, *_: (0, 0)),
            scratch_shapes=[
                pltpu.VMEM((C, D), jnp.float32),
                pltpu.VMEM((C, D), jnp.float32),
                pltpu.SMEM((E * C,), jnp.int32),
                pltpu.SMEM((E,), jnp.int32),
            ],
        ),
        out_shape=jax.ShapeDtypeStruct((T, D), jnp.float32),
        interpret=interpret,
    )(ids, hidden_states, w_gate, w_up, w_down)
    return out


# speedup vs baseline: 1.0520x; 1.0520x over previous
import functools

import jax
import jax.numpy as jnp
from jax import lax
from jax.experimental import pallas as pl
from jax.experimental.pallas import tpu as pltpu
from jax.experimental.pallas import tpu_sc as plsc

T, D, E, F, C = 2048, 768, 64, 2048, 128
RC = 32
NRC = C // RC

_NC, _NS, _L = 2, 16, 16
_NW = 16
_TPW = T // _NW


def _sc_argmax_body(logits_t_hbm, ids_hbm, blk_v, ids_v, sem):
    wid = lax.axis_index("s") * _NC + lax.axis_index("c")

    @pl.when(wid < _NW)
    def _():
        base = wid * _TPW
        pltpu.sync_copy(logits_t_hbm.at[:, pl.ds(base, _TPW)], blk_v)
        for g in range(_TPW // _L):
            best = blk_v[0, pl.ds(g * _L, _L)]
            bid = jnp.zeros((_L,), jnp.int32)
            for e in range(1, E):
                v = blk_v[e, pl.ds(g * _L, _L)]
                upd = v > best
                best = jnp.where(upd, v, best)
                bid = jnp.where(upd, jnp.full((_L,), e, jnp.int32), bid)
            ids_v[pl.ds(g * _L, _L)] = bid
        pltpu.sync_copy(ids_v, ids_hbm.at[pl.ds(base, _TPW)])


_sc_argmax = functools.partial(
    pl.kernel,
    out_type=jax.ShapeDtypeStruct((T,), jnp.int32),
    mesh=plsc.VectorSubcoreMesh(core_axis_name="c", subcore_axis_name="s",
                                num_cores=_NC, num_subcores=_NS),
    scratch_types=[
        pltpu.VMEM((E, _TPW), jnp.float32),
        pltpu.VMEM((_TPW,), jnp.int32),
        pltpu.SemaphoreType.DMA,
    ],
)(_sc_argmax_body)


def _ffn_body(ids_ref, hid_ref, wg_ref, wu_ref, wd_ref, out_ref,
              xb_ref, acc_ref, tfs_ref, cnt_ref):
    e = pl.program_id(0)

    @pl.when(e == 0)
    def _():
        out_ref[...] = jnp.zeros_like(out_ref)

        def init_cnt(i, _):
            cnt_ref[i] = 0
            return 0
        jax.lax.fori_loop(0, E, init_cnt, 0, unroll=True)

        def rank(t, _):
            ex = ids_ref[t]
            p = cnt_ref[ex]

            @pl.when(p < C)
            def _():
                tfs_ref[ex * C + p] = t

            cnt_ref[ex] = p + 1
            return 0
        jax.lax.fori_loop(0, T, rank, 0)

    n = jnp.minimum(cnt_ref[e], C)
    xb_ref[...] = jnp.zeros_like(xb_ref)

    def gather(c, _):
        t = tfs_ref[e * C + c]
        xb_ref[pl.ds(c, 1), :] = hid_ref[pl.ds(t, 1), :]
        return 0
    jax.lax.fori_loop(0, n, gather, 0)

    for k in range(NRC):
        @pl.when(n > k * RC)
        def _(k=k):
            xs = xb_ref[k * RC:(k + 1) * RC, :]
            g = jnp.dot(xs, wg_ref[0], preferred_element_type=jnp.float32)
            u = jnp.dot(xs, wu_ref[0], preferred_element_type=jnp.float32)
            h = g * jax.nn.sigmoid(g) * u
            acc_ref[k * RC:(k + 1) * RC, :] = jnp.dot(
                h, wd_ref[0], preferred_element_type=jnp.float32)

    def scatter(c, _):
        t = tfs_ref[e * C + c]
        out_ref[pl.ds(t, 1), :] = acc_ref[pl.ds(c, 1), :]
        return 0
    jax.lax.fori_loop(0, n, scatter, 0)


@jax.jit
def kernel(hidden_states, router_logits, w_gate, w_up, w_down):
    ids = _sc_argmax(router_logits.T)

    out = pl.pallas_call(
        _ffn_body,
        grid_spec=pltpu.PrefetchScalarGridSpec(
            num_scalar_prefetch=1,
            grid=(E,),
            in_specs=[
                pl.BlockSpec((T, D), lambda e, *_: (0, 0)),
                pl.BlockSpec((1, D, F), lambda e, *_: (e, 0, 0)),
                pl.BlockSpec((1, D, F), lambda e, *_: (e, 0, 0)),
                pl.BlockSpec((1, F, D), lambda e, *_: (e, 0, 0)),
            ],
            out_specs=pl.BlockSpec((T, D), lambda e, *_: (0, 0)),
            scratch_shapes=[
                pltpu.VMEM((C, D), jnp.float32),
                pltpu.VMEM((C, D), jnp.float32),
                pltpu.SMEM((E * C,), jnp.int32),
                pltpu.SMEM((E,), jnp.int32),
            ],
        ),
        out_shape=jax.ShapeDtypeStruct((T, D), jnp.float32),
    )(ids, hidden_states, w_gate, w_up, w_down)
    return out
